# baseline (speedup 1.0000x reference)
import jax
import jax.numpy as jnp
from jax import lax
from jax.experimental import pallas as pl
from jax.experimental.pallas import tpu as pltpu
from jax.experimental.pallas import tpu_sc as plsc

BATCH = 16384
DIM = 128
NC = 2
NS = 16
NW = NC * NS
B_PER_W = BATCH // NW
CHUNK = 128
NCHUNK = B_PER_W // CHUNK
NBUF = 3
LANES = 16
GROUPS = CHUNK // LANES
COL_BLK = 16


def _sc_body(user_id, item_id, user_table, item_table, out,
             idx_u, idx_i, rows_u, rows_i, out_v, sems_u, sems_i):
    wid = lax.axis_index("s") * NC + lax.axis_index("c")
    base = wid * B_PER_W
    lane = lax.iota(jnp.int32, 16)

    cp_u = pltpu.async_copy(user_id.at[pl.ds(base, B_PER_W)], idx_u,
                            sems_u[0])
    cp_i = pltpu.async_copy(item_id.at[pl.ds(base, B_PER_W)], idx_i,
                            sems_i[0])
    cp_u.wait()
    cp_i.wait()

    def gathers(cc, b):
        return (pltpu.make_async_copy(
                    user_table.at[idx_u.at[pl.ds(cc * CHUNK, CHUNK)]],
                    rows_u[b], sems_u[b]),
                pltpu.make_async_copy(
                    item_table.at[idx_i.at[pl.ds(cc * CHUNK, CHUNK)]],
                    rows_i[b], sems_i[b]))

    def start(cc, b):
        gu, gi = gathers(cc, b)
        gu.start()
        gi.start()

    def compute(cc, b):
        ru = rows_u[b]
        ri = rows_i[b]

        @plsc.parallel_loop(0, GROUPS, 1)
        def _(g):
            rowv = g * LANES + lane

            def col_body(cb, acc):
                cbase = cb * COL_BLK + lane
                for t in range(COL_BLK):
                    colv = (cbase + t) & (DIM - 1)
                    u = plsc.load_gather(ru, [rowv, colv])
                    v = plsc.load_gather(ri, [rowv, colv])
                    acc = acc + u * v
                return acc

            acc = lax.fori_loop(0, DIM // COL_BLK, col_body,
                                jnp.zeros((16,), jnp.float32))
            out_v[pl.ds(cc * CHUNK + g * LANES, 16)] = acc

    for cc in range(min(NBUF, NCHUNK)):
        start(cc, cc)
    for cc in range(NCHUNK):
        b = cc % NBUF
        gu, gi = gathers(cc, b)
        gu.wait()
        gi.wait()
        compute(cc, b)
        if cc + NBUF < NCHUNK:
            start(cc + NBUF, b)

    pltpu.sync_copy(out_v, out.at[pl.ds(base, B_PER_W)])


@jax.jit
def kernel(user_id, item_id, user_table, item_table):
    mesh = plsc.VectorSubcoreMesh(
        core_axis_name="c", subcore_axis_name="s",
        num_cores=NC, num_subcores=NS)
    run = pl.kernel(
        _sc_body,
        out_type=jax.ShapeDtypeStruct((BATCH,), jnp.float32),
        mesh=mesh,
        compiler_params=pltpu.CompilerParams(needs_layout_passes=False),
        scratch_types=[
            pltpu.VMEM((B_PER_W,), jnp.int32),
            pltpu.VMEM((B_PER_W,), jnp.int32),
            [pltpu.VMEM((CHUNK, DIM), jnp.float32) for _ in range(NBUF)],
            [pltpu.VMEM((CHUNK, DIM), jnp.float32) for _ in range(NBUF)],
            pltpu.VMEM((B_PER_W,), jnp.float32),
            [pltpu.SemaphoreType.DMA for _ in range(NBUF)],
            [pltpu.SemaphoreType.DMA for _ in range(NBUF)],
        ],
    )
    return run(user_id, item_id, user_table, item_table)


# speedup vs baseline: 1.0104x; 1.0104x over previous
import jax
import jax.numpy as jnp
from jax import lax
from jax.experimental import pallas as pl
from jax.experimental.pallas import tpu as pltpu
from jax.experimental.pallas import tpu_sc as plsc

BATCH = 16384
DIM = 128
NC = 2
NS = 16
NW = NC * NS
B_PER_W = BATCH // NW
CHUNK = 128
NCHUNK = B_PER_W // CHUNK
LANES = 16
GROUPS = CHUNK // LANES
COL_BLK = 16


def _sc_body(user_id, item_id, user_table, item_table, out,
             idx_u, idx_i, rows_u, rows_i, out_v,
             sem_u0, sem_u1, sem_i0, sem_i1):
    wid = lax.axis_index("s") * NC + lax.axis_index("c")
    base = wid * B_PER_W
    lane = lax.iota(jnp.int32, 16)

    cp_u = pltpu.async_copy(user_id.at[pl.ds(base, B_PER_W)], idx_u, sem_u0)
    cp_i = pltpu.async_copy(item_id.at[pl.ds(base, B_PER_W)], idx_i, sem_i0)
    cp_u.wait()
    cp_i.wait()

    def gathers(cc, b):
        sem_u = sem_u0 if b == 0 else sem_u1
        sem_i = sem_i0 if b == 0 else sem_i1
        return (pltpu.make_async_copy(
                    user_table.at[idx_u.at[pl.ds(cc * CHUNK, CHUNK)]],
                    rows_u.at[pl.ds(b * CHUNK, CHUNK)], sem_u),
                pltpu.make_async_copy(
                    item_table.at[idx_i.at[pl.ds(cc * CHUNK, CHUNK)]],
                    rows_i.at[pl.ds(b * CHUNK, CHUNK)], sem_i))

    def start(cc, b):
        gu, gi = gathers(cc, b)
        gu.start()
        gi.start()

    def wait(cc, b):
        gu, gi = gathers(cc, b)
        gu.wait()
        gi.wait()

    start(0, 0)
    start(1, 1)

    def chunk_body(cc, _):
        par = lax.rem(cc, 2)
        rowbase = par * CHUNK

        @pl.when(par == 0)
        def _():
            wait(cc, 0)

        @pl.when(par == 1)
        def _():
            wait(cc, 1)

        @plsc.parallel_loop(0, GROUPS, 1)
        def _(g):
            rowv = rowbase + g * LANES + lane

            def col_body(cb, acc):
                cbase = cb * COL_BLK + lane
                for t in range(COL_BLK):
                    colv = (cbase + t) & (DIM - 1)
                    u = plsc.load_gather(rows_u, [rowv, colv])
                    v = plsc.load_gather(rows_i, [rowv, colv])
                    acc = acc + u * v
                return acc

            acc = lax.fori_loop(0, DIM // COL_BLK, col_body,
                                jnp.zeros((16,), jnp.float32))
            out_v[pl.ds(cc * CHUNK + g * LANES, 16)] = acc

        @pl.when((par == 0) & (cc + 2 < NCHUNK))
        def _():
            start(cc + 2, 0)

        @pl.when((par == 1) & (cc + 2 < NCHUNK))
        def _():
            start(cc + 2, 1)

        return 0

    lax.fori_loop(0, NCHUNK, chunk_body, 0)

    pltpu.sync_copy(out_v, out.at[pl.ds(base, B_PER_W)])


@jax.jit
def kernel(user_id, item_id, user_table, item_table):
    mesh = plsc.VectorSubcoreMesh(
        core_axis_name="c", subcore_axis_name="s",
        num_cores=NC, num_subcores=NS)
    run = pl.kernel(
        _sc_body,
        out_type=jax.ShapeDtypeStruct((BATCH,), jnp.float32),
        mesh=mesh,
        compiler_params=pltpu.CompilerParams(needs_layout_passes=False),
        scratch_types=[
            pltpu.VMEM((B_PER_W,), jnp.int32),
            pltpu.VMEM((B_PER_W,), jnp.int32),
            pltpu.VMEM((2 * CHUNK, DIM), jnp.float32),
            pltpu.VMEM((2 * CHUNK, DIM), jnp.float32),
            pltpu.VMEM((B_PER_W,), jnp.float32),
            pltpu.SemaphoreType.DMA,
            pltpu.SemaphoreType.DMA,
            pltpu.SemaphoreType.DMA,
            pltpu.SemaphoreType.DMA,
        ],
    )
    return run(user_id, item_id, user_table, item_table)


# speedup vs baseline: 1.0139x; 1.0034x over previous
import jax
import jax.numpy as jnp
from jax import lax
from jax.experimental import pallas as pl
from jax.experimental.pallas import tpu as pltpu
from jax.experimental.pallas import tpu_sc as plsc

BATCH = 16384
DIM = 128
NC = 2
NS = 16
NW = NC * NS
B_PER_W = BATCH // NW
CHUNK = 128
NCHUNK = B_PER_W // CHUNK
LANES = 16
GROUPS = CHUNK // LANES
COL_BLK = 16


def _sc_body(user_id, item_id, user_table, item_table, out,
             idx_u, idx_i, rows_u, rows_i, out_v,
             sem_u0, sem_u1, sem_i0, sem_i1):
    wid = lax.axis_index("s") * NC + lax.axis_index("c")
    base = wid * B_PER_W
    lane = lax.iota(jnp.int32, 16)

    cp_u = pltpu.async_copy(user_id.at[pl.ds(base, B_PER_W)], idx_u, sem_u0)
    cp_i = pltpu.async_copy(item_id.at[pl.ds(base, B_PER_W)], idx_i, sem_i0)
    cp_u.wait()
    cp_i.wait()

    def gathers(cc, b):
        sem_u = sem_u0 if b == 0 else sem_u1
        sem_i = sem_i0 if b == 0 else sem_i1
        return (pltpu.make_async_copy(
                    user_table.at[idx_u.at[pl.ds(cc * CHUNK, CHUNK)]],
                    rows_u.at[pl.ds(b * CHUNK, CHUNK)], sem_u),
                pltpu.make_async_copy(
                    item_table.at[idx_i.at[pl.ds(cc * CHUNK, CHUNK)]],
                    rows_i.at[pl.ds(b * CHUNK, CHUNK)], sem_i))

    def start(cc, b):
        gu, gi = gathers(cc, b)
        gu.start()
        gi.start()

    def wait(cc, b):
        gu, gi = gathers(cc, b)
        gu.wait()
        gi.wait()

    start(0, 0)
    start(1, 1)

    def chunk_body(cc, _):
        par = lax.rem(cc, 2)
        rowbase = par * CHUNK

        @pl.when(par == 0)
        def _():
            wait(cc, 0)

        @pl.when(par == 1)
        def _():
            wait(cc, 1)

        @plsc.parallel_loop(0, GROUPS, 1)
        def _(g):
            rowv = rowbase + g * LANES + lane

            def col_body(cb, acc):
                cbase = cb * COL_BLK + lane
                for t in range(COL_BLK):
                    colv = (cbase + t) & (DIM - 1)
                    u = plsc.load_gather(rows_u, [rowv, colv])
                    v = plsc.load_gather(rows_i, [rowv, colv])
                    acc = acc + u * v
                return acc

            acc = lax.fori_loop(0, DIM // COL_BLK, col_body,
                                jnp.zeros((16,), jnp.float32))
            out_v[pl.ds(cc * CHUNK + g * LANES, 16)] = acc

        @pl.when((par == 0) & (cc + 2 < NCHUNK))
        def _():
            start(cc + 2, 0)

        @pl.when((par == 1) & (cc + 2 < NCHUNK))
        def _():
            start(cc + 2, 1)

        return 0

    lax.fori_loop(0, NCHUNK, chunk_body, 0)

    pltpu.sync_copy(out_v, out.at[pl.ds(base, B_PER_W)])


@jax.jit
def kernel(user_id, item_id, user_table, item_table):
    mesh = plsc.VectorSubcoreMesh(
        core_axis_name="c", subcore_axis_name="s",
        num_cores=NC, num_subcores=NS)
    run = pl.kernel(
        _sc_body,
        out_type=jax.ShapeDtypeStruct((BATCH,), jnp.float32),
        mesh=mesh,
        compiler_params=pltpu.CompilerParams(needs_layout_passes=False, skip_device_barrier=True),
        scratch_types=[
            pltpu.VMEM((B_PER_W,), jnp.int32),
            pltpu.VMEM((B_PER_W,), jnp.int32),
            pltpu.VMEM((2 * CHUNK, DIM), jnp.float32),
            pltpu.VMEM((2 * CHUNK, DIM), jnp.float32),
            pltpu.VMEM((B_PER_W,), jnp.float32),
            pltpu.SemaphoreType.DMA,
            pltpu.SemaphoreType.DMA,
            pltpu.SemaphoreType.DMA,
            pltpu.SemaphoreType.DMA,
        ],
    )
    return run(user_id, item_id, user_table, item_table)
